# Initial kernel scaffold; baseline (speedup 1.0000x reference)
import jax
import jax.numpy as jnp
from jax.experimental import pallas as pl


def kernel(x, W, b):
    raise NotImplementedError("write your pallas kernel here")


# speedup vs baseline: 1.2685x; 1.2685x over previous
import functools

import jax
import jax.numpy as jnp
from jax import lax
from jax.experimental import pallas as pl
from jax.experimental.pallas import tpu as pltpu

D_MODEL = 4096
NUM_EXPERTS = 64
TOP_K = 8
CAPACITY_FACTOR = 1.25
ALPHA = 0.01

BLK = 512


def _gate_kernel(x_ref, wt_ref, b_ref, idx_ref, w_ref, aux_ref, load_acc,
                 *, n_steps, n_tokens):
    i = pl.program_id(0)

    logits = jnp.dot(x_ref[...], wt_ref[...],
                     preferred_element_type=jnp.float32) + b_ref[...]

    m = jnp.max(logits, axis=-1, keepdims=True)
    e = jnp.exp(logits - m)
    s = jnp.sum(e, axis=-1, keepdims=True)
    p = e / s

    col = lax.broadcasted_iota(jnp.int32, (BLK, NUM_EXPERTS), 1)

    idx_cols = []
    val_cols = []
    pm = p
    for _ in range(TOP_K):
        mv = jnp.max(pm, axis=-1, keepdims=True)
        is_max = pm == mv
        sel = jnp.min(jnp.where(is_max, col, NUM_EXPERTS), axis=-1,
                      keepdims=True)
        idx_cols.append(sel)
        val_cols.append(mv)
        pm = jnp.where(col == sel, -1.0, pm)

    topk_idx = jnp.concatenate(idx_cols, axis=-1)
    topk_val = jnp.concatenate(val_cols, axis=-1)
    denom = jnp.sum(topk_val, axis=-1, keepdims=True) + 1e-9
    idx_ref[...] = topk_idx
    w_ref[...] = topk_val / denom

    sel_mask = (pm < 0.0).astype(jnp.float32)
    load_part = jnp.sum(sel_mask, axis=0, keepdims=True)

    @pl.when(i == 0)
    def _init():
        load_acc[...] = jnp.zeros_like(load_acc)

    load_acc[...] += load_part

    @pl.when(i == n_steps - 1)
    def _finalize():
        load = load_acc[...]
        capacity = CAPACITY_FACTOR * (n_tokens * TOP_K) / NUM_EXPERTS
        penalty = jnp.sum(jnp.maximum(load - capacity, 0.0))
        aux = ALPHA * penalty / NUM_EXPERTS / n_tokens
        aux_ref[...] = aux.reshape(1, 1)


def kernel(x, W, b):
    batch, seq, d_model = x.shape
    n_tokens = batch * seq
    xf = x.reshape(n_tokens, d_model)
    wt = W.T
    n_steps = n_tokens // BLK

    idx, w, aux = pl.pallas_call(
        functools.partial(_gate_kernel, n_steps=n_steps, n_tokens=n_tokens),
        grid=(n_steps,),
        in_specs=[
            pl.BlockSpec((BLK, d_model), lambda i: (i, 0)),
            pl.BlockSpec((d_model, NUM_EXPERTS), lambda i: (0, 0)),
            pl.BlockSpec((NUM_EXPERTS,), lambda i: (0,)),
        ],
        out_specs=[
            pl.BlockSpec((BLK, TOP_K), lambda i: (i, 0)),
            pl.BlockSpec((BLK, TOP_K), lambda i: (i, 0)),
            pl.BlockSpec((1, 1), lambda i: (0, 0)),
        ],
        out_shape=[
            jax.ShapeDtypeStruct((n_tokens, TOP_K), jnp.int32),
            jax.ShapeDtypeStruct((n_tokens, TOP_K), jnp.float32),
            jax.ShapeDtypeStruct((1, 1), jnp.float32),
        ],
        scratch_shapes=[pltpu.VMEM((1, NUM_EXPERTS), jnp.float32)],
    )(xf, wt, b)

    return (idx.reshape(batch, seq, TOP_K),
            w.reshape(batch, seq, TOP_K),
            aux[0, 0])
